# Initial kernel scaffold; baseline (speedup 1.0000x reference)
import jax
import jax.numpy as jnp
from jax.experimental import pallas as pl


def kernel(x, donor_labels, W, b):
    raise NotImplementedError("write your pallas kernel here")


# speedup vs baseline: 1.6140x; 1.6140x over previous
import jax
import jax.numpy as jnp
from jax.experimental import pallas as pl
from jax.experimental.pallas import tpu as pltpu


def _dense_body(labels_ref, x_ref, w_ref, b_ref, out_ref):
    n_donors = w_ref.shape[0]
    labels = labels_ref[...]
    acc = jnp.zeros(out_ref.shape, dtype=jnp.float32)
    for d in range(n_donors):
        pred = jax.lax.dot_general(
            x_ref[...], w_ref[d],
            dimension_numbers=(((1,), (1,)), ((), ())),
            preferred_element_type=jnp.float32,
        ) + b_ref[d][None, :]
        acc = jnp.where(labels == d, pred, acc)
    out_ref[...] = acc


def kernel(x, donor_labels, W, b):
    B, input_dim = x.shape
    n_donors, n_genes, _ = W.shape
    TOK_BLK = 256
    n_blocks = B // TOK_BLK
    labels2 = donor_labels.reshape(B, 1)
    return pl.pallas_call(
        _dense_body,
        grid=(n_blocks,),
        in_specs=[
            pl.BlockSpec((TOK_BLK, 1), lambda i: (i, 0)),
            pl.BlockSpec((TOK_BLK, input_dim), lambda i: (i, 0)),
            pl.BlockSpec((n_donors, n_genes, input_dim), lambda i: (0, 0, 0)),
            pl.BlockSpec((n_donors, n_genes), lambda i: (0, 0)),
        ],
        out_specs=pl.BlockSpec((TOK_BLK, n_genes), lambda i: (i, 0)),
        out_shape=jax.ShapeDtypeStruct((B, n_genes), x.dtype),
    )(labels2, x, W, b)
